# Initial kernel scaffold; baseline (speedup 1.0000x reference)
import jax
import jax.numpy as jnp
from jax.experimental import pallas as pl


def kernel(hidden_states, Wq, Wk, Wv, Wg, Wo):
    raise NotImplementedError("write your pallas kernel here")


# speedup vs baseline: 1.5769x; 1.5769x over previous
import jax
import jax.numpy as jnp
from jax.experimental import pallas as pl

B_, T_, D_ = 1, 2048, 2048
H_, HKV_, HD_ = 16, 4, 128
BS_, K_, W_ = 64, 16, 512
NB_ = T_ // BS_
G_ = H_ // HKV_
NEG_ = -1e30
SCALE_ = HD_ ** -0.5
QB_ = 256
NQ_ = T_ // QB_
PPAD_ = 3200


def _msoftmax(s, mask):
    s = jnp.where(mask, s, NEG_)
    m = jnp.max(s, axis=-1, keepdims=True)
    e = jnp.where(mask, jnp.exp(s - m), 0.0)
    d = jnp.sum(e, axis=-1, keepdims=True)
    return e / jnp.maximum(d, 1e-20)


def _proj_body(x_ref, w_ref, q_ref, k_ref, v_ref, g_ref):
    y = jax.lax.dot_general(x_ref[...], w_ref[...], (((1,), (0,)), ((), ())),
                            preferred_element_type=jnp.float32)
    q_ref[...] = y[:, :2048]
    k_ref[...] = y[:, 2048:2560]
    v_ref[...] = y[:, 2560:3072]
    g_ref[...] = y[:, 3072:3200]


def _pool_body(k_ref, v_ref, kb_ref, vb_ref):
    kb_ref[...] = jnp.mean(k_ref[...].reshape(NB_, BS_, HKV_ * HD_), axis=1)
    vb_ref[...] = jnp.mean(v_ref[...].reshape(NB_, BS_, HKV_ * HD_), axis=1)


def _cmp_body(q_ref, kb_ref, vb_ref, ocmp_ref, sel_ref):
    i = pl.program_id(0)
    t = i * QB_ + jax.lax.broadcasted_iota(jnp.int32, (QB_, NB_), 0)
    n = jax.lax.broadcasted_iota(jnp.int32, (QB_, NB_), 1)
    m_cmp = ((n + 1) * BS_ - 1) <= t
    force = (n == (t // BS_)) | (n == 0)
    jj = jax.lax.broadcasted_iota(jnp.int32, (NB_, NB_), 1)
    ii = jax.lax.broadcasted_iota(jnp.int32, (NB_, NB_), 0)
    tie = (jj < ii)[None]
    for hk in range(HKV_):
        kb = kb_ref[:, hk * HD_:(hk + 1) * HD_]
        vb = vb_ref[:, hk * HD_:(hk + 1) * HD_]
        imp = jnp.zeros((QB_, NB_), jnp.float32)
        for g in range(G_):
            h = hk * G_ + g
            qh = q_ref[:, h * HD_:(h + 1) * HD_] * SCALE_
            s = jax.lax.dot_general(qh, kb, (((1,), (1,)), ((), ())),
                                    preferred_element_type=jnp.float32)
            p = _msoftmax(s, m_cmp)
            o = jax.lax.dot_general(p, vb, (((1,), (0,)), ((), ())),
                                    preferred_element_type=jnp.float32)
            ocmp_ref[:, h * HD_:(h + 1) * HD_] = o
            imp = imp + p
        imp = imp + jnp.where(force, 1e9, 0.0)
        beats = (imp[:, None, :] > imp[:, :, None]) | (
            (imp[:, None, :] == imp[:, :, None]) & tie)
        rank = jnp.sum(beats.astype(jnp.float32), axis=-1)
        sel_ref[:, hk * NB_:(hk + 1) * NB_] = (rank < K_).astype(jnp.float32)


def _attn_body(q_ref, k_ref, v_ref, g_ref, ocmp_ref, sel_ref, o_ref):
    i = pl.program_id(0)
    t = i * QB_ + jax.lax.broadcasted_iota(jnp.int32, (QB_, T_), 0)
    sc = jax.lax.broadcasted_iota(jnp.int32, (QB_, T_), 1)
    causal = sc <= t
    m_swa = causal & (sc > t - W_)
    en = jax.lax.broadcasted_iota(jnp.int32, (NB_, T_), 0)
    es = jax.lax.broadcasted_iota(jnp.int32, (NB_, T_), 1)
    expand = ((es // BS_) == en).astype(jnp.float32)
    gates = jax.nn.sigmoid(g_ref[:, :H_ * 3])
    for hk in range(HKV_):
        kk = k_ref[:, hk * HD_:(hk + 1) * HD_]
        vv = v_ref[:, hk * HD_:(hk + 1) * HD_]
        selc = sel_ref[:, hk * NB_:(hk + 1) * NB_]
        selexp = jax.lax.dot_general(selc, expand, (((1,), (0,)), ((), ())),
                                     preferred_element_type=jnp.float32)
        m_slc = (selexp > 0.5) & causal
        for g in range(G_):
            h = hk * G_ + g
            qh = q_ref[:, h * HD_:(h + 1) * HD_] * SCALE_
            s = jax.lax.dot_general(qh, kk, (((1,), (1,)), ((), ())),
                                    preferred_element_type=jnp.float32)
            p_slc = _msoftmax(s, m_slc)
            p_swa = _msoftmax(s, m_swa)
            o_slc = jax.lax.dot_general(p_slc, vv, (((1,), (0,)), ((), ())),
                                        preferred_element_type=jnp.float32)
            o_swa = jax.lax.dot_general(p_swa, vv, (((1,), (0,)), ((), ())),
                                        preferred_element_type=jnp.float32)
            ocmp_h = ocmp_ref[:, h * HD_:(h + 1) * HD_]
            gc = gates[:, 3 * h:3 * h + 1]
            gs = gates[:, 3 * h + 1:3 * h + 2]
            gw = gates[:, 3 * h + 2:3 * h + 3]
            o_ref[:, h * HD_:(h + 1) * HD_] = ocmp_h * gc + o_slc * gs + o_swa * gw


def _oproj_body(z_ref, w_ref, o_ref):
    o_ref[...] = jax.lax.dot_general(z_ref[...], w_ref[...], (((1,), (0,)), ((), ())),
                                     preferred_element_type=jnp.float32)


def _nsa_pallas(x, WcatT, WoT, interpret=False):
    f32 = jnp.float32
    q, k, v, g = pl.pallas_call(
        _proj_body,
        grid=(NQ_,),
        in_specs=[
            pl.BlockSpec((QB_, D_), lambda i: (i, 0)),
            pl.BlockSpec((D_, PPAD_), lambda i: (0, 0)),
        ],
        out_specs=[
            pl.BlockSpec((QB_, 2048), lambda i: (i, 0)),
            pl.BlockSpec((QB_, 512), lambda i: (i, 0)),
            pl.BlockSpec((QB_, 512), lambda i: (i, 0)),
            pl.BlockSpec((QB_, 128), lambda i: (i, 0)),
        ],
        out_shape=[
            jax.ShapeDtypeStruct((T_, 2048), f32),
            jax.ShapeDtypeStruct((T_, 512), f32),
            jax.ShapeDtypeStruct((T_, 512), f32),
            jax.ShapeDtypeStruct((T_, 128), f32),
        ],
        interpret=interpret,
    )(x, WcatT)

    kb, vb = pl.pallas_call(
        _pool_body,
        out_shape=[
            jax.ShapeDtypeStruct((NB_, HKV_ * HD_), f32),
            jax.ShapeDtypeStruct((NB_, HKV_ * HD_), f32),
        ],
        interpret=interpret,
    )(k, v)

    ocmp, sel = pl.pallas_call(
        _cmp_body,
        grid=(NQ_,),
        in_specs=[
            pl.BlockSpec((QB_, 2048), lambda i: (i, 0)),
            pl.BlockSpec((NB_, HKV_ * HD_), lambda i: (0, 0)),
            pl.BlockSpec((NB_, HKV_ * HD_), lambda i: (0, 0)),
        ],
        out_specs=[
            pl.BlockSpec((QB_, 2048), lambda i: (i, 0)),
            pl.BlockSpec((QB_, HKV_ * NB_), lambda i: (i, 0)),
        ],
        out_shape=[
            jax.ShapeDtypeStruct((T_, 2048), f32),
            jax.ShapeDtypeStruct((T_, HKV_ * NB_), f32),
        ],
        interpret=interpret,
    )(q, kb, vb)

    z = pl.pallas_call(
        _attn_body,
        grid=(NQ_,),
        in_specs=[
            pl.BlockSpec((QB_, 2048), lambda i: (i, 0)),
            pl.BlockSpec((T_, 512), lambda i: (0, 0)),
            pl.BlockSpec((T_, 512), lambda i: (0, 0)),
            pl.BlockSpec((QB_, 128), lambda i: (i, 0)),
            pl.BlockSpec((QB_, 2048), lambda i: (i, 0)),
            pl.BlockSpec((QB_, HKV_ * NB_), lambda i: (i, 0)),
        ],
        out_specs=pl.BlockSpec((QB_, 2048), lambda i: (i, 0)),
        out_shape=jax.ShapeDtypeStruct((T_, 2048), f32),
        interpret=interpret,
    )(q, k, v, g, ocmp, sel)

    out = pl.pallas_call(
        _oproj_body,
        grid=(NQ_,),
        in_specs=[
            pl.BlockSpec((QB_, 2048), lambda i: (i, 0)),
            pl.BlockSpec((D_, D_), lambda i: (0, 0)),
        ],
        out_specs=pl.BlockSpec((QB_, D_), lambda i: (i, 0)),
        out_shape=jax.ShapeDtypeStruct((T_, D_), f32),
        interpret=interpret,
    )(z, WoT)
    return out


def kernel(hidden_states, Wq, Wk, Wv, Wg, Wo):
    x = hidden_states.reshape(T_, D_)
    Wcat = jnp.concatenate([Wq, Wk, Wv,
                            jnp.pad(Wg, ((0, PPAD_ - 3072 - H_ * 3), (0, 0)))], axis=0)
    out = _nsa_pallas(x, Wcat.T, Wo.T)
    return out.reshape(B_, T_, D_)
